# baseline (speedup 1.0000x reference)
import functools

import jax
import jax.numpy as jnp
from jax import lax
from jax.experimental import pallas as pl
from jax.experimental.pallas import tpu as pltpu
from jax.experimental.pallas import tpu_sc as plsc

_L = 16


def _sc_body(h, w, d, n_rows, x_hbm, row_hbm, col_hbm, out_hbm,
             posbuf, xbuf, colbuf, rowbuf, sem):
    c2 = 2 * d
    info = plsc.get_sparse_core_info()
    wid = lax.axis_index("s") * info.num_cores + lax.axis_index("c")

    pltpu.sync_copy(col_hbm.at[pl.ds(0, w)], colbuf)
    pltpu.sync_copy(row_hbm.at[wid], rowbuf)

    def fill(v, carry):
        j = v // (d // _L)
        c = (v % (d // _L)) * _L
        posbuf[j, pl.ds(c, _L)] = colbuf[j, pl.ds(c, _L)]
        posbuf[j, pl.ds(d + c, _L)] = rowbuf[pl.ds(c, _L)]
        return carry

    lax.fori_loop(0, w * (d // _L), fill, 0)

    def step(b, carry):
        off = b * h * w + wid * w
        pltpu.sync_copy(x_hbm.at[pl.ds(off, w)], xbuf)

        def add(v, c2_):
            j = v // (c2 // _L)
            c = (v % (c2 // _L)) * _L
            xbuf[j, pl.ds(c, _L)] = xbuf[j, pl.ds(c, _L)] + posbuf[j, pl.ds(c, _L)]
            return c2_

        lax.fori_loop(0, w * (c2 // _L), add, 0)
        pltpu.sync_copy(xbuf, out_hbm.at[pl.ds(off, w)])
        return carry

    lax.fori_loop(0, n_rows // (h * w), step, 0)


def kernel(x, row_embed, col_embed):
    b, c2, h, w = x.shape
    d = c2 // 2
    n_rows = b * h * w
    x2 = jnp.transpose(x, (0, 2, 3, 1)).reshape(n_rows, c2)

    scmesh = plsc.VectorSubcoreMesh(core_axis_name="c", subcore_axis_name="s")
    sc_add = functools.partial(
        pl.kernel,
        out_type=jax.ShapeDtypeStruct((n_rows, c2), x.dtype),
        mesh=scmesh,
        scratch_types=[
            pltpu.VMEM((w, c2), x.dtype),
            pltpu.VMEM((w, c2), x.dtype),
            pltpu.VMEM((w, d), x.dtype),
            pltpu.VMEM((d,), x.dtype),
            pltpu.SemaphoreType.DMA,
        ],
    )(functools.partial(_sc_body, h, w, d, n_rows))

    out2 = sc_add(x2, row_embed, col_embed)
    return jnp.transpose(out2.reshape(b, h, w, c2), (0, 3, 1, 2))


# speedup vs baseline: 1.5808x; 1.5808x over previous
import functools

import jax
import jax.numpy as jnp
from jax import lax
from jax.experimental import pallas as pl
from jax.experimental.pallas import tpu as pltpu
from jax.experimental.pallas import tpu_sc as plsc

_L = 16


def _sc_body(h, w, d, n_rows, x_hbm, row_hbm, col_hbm, out_hbm,
             posbuf, xbuf, colbuf, rowbuf, sem):
    c2 = 2 * d
    info = plsc.get_sparse_core_info()
    wid = lax.axis_index("s") * info.num_cores + lax.axis_index("c")

    pltpu.sync_copy(col_hbm.at[pl.ds(0, w)], colbuf)
    pltpu.sync_copy(row_hbm.at[wid], rowbuf)

    def fill(v, carry):
        j = v // (d // _L)
        c = (v % (d // _L)) * _L
        posbuf[j, pl.ds(c, _L)] = colbuf[j, pl.ds(c, _L)]
        posbuf[j, pl.ds(d + c, _L)] = rowbuf[pl.ds(c, _L)]
        return carry

    lax.fori_loop(0, w * (d // _L), fill, 0)

    def step(b, carry):
        off = b * h * w + wid * w
        pltpu.sync_copy(x_hbm.at[pl.ds(off, w)], xbuf)

        def add(j, c2_):
            for cc in range(c2 // _L):
                c = cc * _L
                xbuf[j, pl.ds(c, _L)] = (
                    xbuf[j, pl.ds(c, _L)] + posbuf[j, pl.ds(c, _L)]
                )
            return c2_

        lax.fori_loop(0, w, add, 0)
        pltpu.sync_copy(xbuf, out_hbm.at[pl.ds(off, w)])
        return carry

    lax.fori_loop(0, n_rows // (h * w), step, 0)


def kernel(x, row_embed, col_embed):
    b, c2, h, w = x.shape
    d = c2 // 2
    n_rows = b * h * w
    x2 = jnp.transpose(x, (0, 2, 3, 1)).reshape(n_rows, c2)

    scmesh = plsc.VectorSubcoreMesh(core_axis_name="c", subcore_axis_name="s")
    sc_add = functools.partial(
        pl.kernel,
        out_type=jax.ShapeDtypeStruct((n_rows, c2), x.dtype),
        mesh=scmesh,
        scratch_types=[
            pltpu.VMEM((w, c2), x.dtype),
            pltpu.VMEM((w, c2), x.dtype),
            pltpu.VMEM((w, d), x.dtype),
            pltpu.VMEM((d,), x.dtype),
            pltpu.SemaphoreType.DMA,
        ],
    )(functools.partial(_sc_body, h, w, d, n_rows))

    out2 = sc_add(x2, row_embed, col_embed)
    return jnp.transpose(out2.reshape(b, h, w, c2), (0, 3, 1, 2))


# speedup vs baseline: 3.5241x; 2.2294x over previous
import functools

import jax
import jax.numpy as jnp
from jax import lax
from jax.experimental import pallas as pl
from jax.experimental.pallas import tpu as pltpu
from jax.experimental.pallas import tpu_sc as plsc

_L = 16
_B_BLK = 8


def _sc_pos_body(h, w, d, row_hbm, col_hbm, pos_hbm, posbuf, colbuf, rowbuf):
    info = plsc.get_sparse_core_info()
    wid = lax.axis_index("s") * info.num_cores + lax.axis_index("c")

    pltpu.sync_copy(col_hbm.at[pl.ds(0, w)], colbuf)
    pltpu.sync_copy(row_hbm.at[wid], rowbuf)

    def fill(v, carry):
        j = v // (d // _L)
        c = (v % (d // _L)) * _L
        posbuf[j, pl.ds(c, _L)] = colbuf[j, pl.ds(c, _L)]
        posbuf[j, pl.ds(d + c, _L)] = rowbuf[pl.ds(c, _L)]
        return carry

    lax.fori_loop(0, w * (d // _L), fill, 0)
    pltpu.sync_copy(posbuf, pos_hbm.at[pl.ds(wid * w, w)])


def _tc_body(x_ref, pos_ref, out_ref):
    n, h, w, c2 = x_ref.shape
    pos = jnp.reshape(pos_ref[...], (h, w, c2))
    out_ref[...] = x_ref[...] + pos[None]


def kernel(x, row_embed, col_embed):
    b, c2, h, w = x.shape
    d = c2 // 2
    xt = jnp.transpose(x, (0, 2, 3, 1))

    scmesh = plsc.VectorSubcoreMesh(core_axis_name="c", subcore_axis_name="s")
    sc_pos = functools.partial(
        pl.kernel,
        out_type=jax.ShapeDtypeStruct((h * w, c2), x.dtype),
        mesh=scmesh,
        scratch_types=[
            pltpu.VMEM((w, c2), x.dtype),
            pltpu.VMEM((w, d), x.dtype),
            pltpu.VMEM((d,), x.dtype),
        ],
    )(functools.partial(_sc_pos_body, h, w, d))
    pos2 = sc_pos(row_embed, col_embed)

    out = pl.pallas_call(
        _tc_body,
        grid=(b // _B_BLK,),
        in_specs=[
            pl.BlockSpec((_B_BLK, h, w, c2), lambda g: (g, 0, 0, 0)),
            pl.BlockSpec((h * w, c2), lambda g: (0, 0)),
        ],
        out_specs=pl.BlockSpec((_B_BLK, h, w, c2), lambda g: (g, 0, 0, 0)),
        out_shape=jax.ShapeDtypeStruct((b, h, w, c2), x.dtype),
    )(xt, pos2)
    return jnp.transpose(out, (0, 3, 1, 2))


# speedup vs baseline: 5.4497x; 1.5464x over previous
import jax
import jax.numpy as jnp
from jax.experimental import pallas as pl
from jax.experimental.pallas import tpu as pltpu

_B_BLK = 8


def _body(x_ref, row_ref, col_ref, out_ref, pos_ref):
    h = x_ref.shape[1]
    w = x_ref.shape[2]
    d = col_ref.shape[1]

    @pl.when(pl.program_id(0) == 0)
    def _build_pos():
        col = col_ref[0:w, :]
        row = row_ref[0:h, :]
        pos_col = jnp.broadcast_to(col[None, :, :], (h, w, d))
        pos_row = jnp.broadcast_to(row[:, None, :], (h, w, d))
        pos_ref[...] = jnp.concatenate([pos_col, pos_row], axis=-1)

    out_ref[...] = x_ref[...] + pos_ref[...][None]


def kernel(x, row_embed, col_embed):
    b, c2, h, w = x.shape
    xt = jnp.transpose(x, (0, 2, 3, 1))
    grid = (b // _B_BLK,)
    out = pl.pallas_call(
        _body,
        grid=grid,
        in_specs=[
            pl.BlockSpec((_B_BLK, h, w, c2), lambda g: (g, 0, 0, 0)),
            pl.BlockSpec(row_embed.shape, lambda g: (0, 0)),
            pl.BlockSpec(col_embed.shape, lambda g: (0, 0)),
        ],
        out_specs=pl.BlockSpec((_B_BLK, h, w, c2), lambda g: (g, 0, 0, 0)),
        out_shape=jax.ShapeDtypeStruct((b, h, w, c2), x.dtype),
        scratch_shapes=[pltpu.VMEM((h, w, c2), x.dtype)],
    )(xt, row_embed, col_embed)
    return jnp.transpose(out, (0, 3, 1, 2))
